# Initial kernel scaffold; baseline (speedup 1.0000x reference)
import jax
import jax.numpy as jnp
from jax.experimental import pallas as pl


def kernel(x, edge_index, edge_weight, W1, b1, gamma1, beta1, W2, b2, gamma2, beta2):
    raise NotImplementedError("write your pallas kernel here")


# speedup vs baseline: 12.9557x; 12.9557x over previous
import functools

import jax
import jax.numpy as jnp
from jax import lax
from jax.experimental import pallas as pl
from jax.experimental.pallas import tpu as pltpu
from jax.experimental.pallas import tpu_sc as plsc

_N = 10000
_E = 320000
_D = 128
_NC = 2
_NS = 16
_NW = _NC * _NS
_EPT = _E // _NW
_CH = 80
_NCH = _EPT // _CH
_RPT = _N // _NS
_BLK = 1000
_NB = _N // _BLK

@functools.lru_cache(maxsize=None)
def _sc_mesh():
    return plsc.VectorSubcoreMesh(core_axis_name="c", subcore_axis_name="s",
                                  num_cores=_NC, num_subcores=_NS)


def _sc_degree(dstT, wT):
    @functools.partial(
        pl.kernel,
        out_type=jax.ShapeDtypeStruct((_NW, _RPT, 16), jnp.float32),
        mesh=_sc_mesh(),
        compiler_params=pltpu.CompilerParams(needs_layout_passes=False, use_tc_tiling_on_sc=False),
        scratch_types=[
            pltpu.VMEM_SHARED((_N, 16), jnp.float32),
            pltpu.VMEM((_NCH, _CH), jnp.int32),
            pltpu.VMEM((_EPT,), jnp.float32),
            pltpu.VMEM((_CH, 16), jnp.float32),
        ],
    )
    def k(dst_hbm, w_hbm, out_hbm, acc, dstb, wb, stage):
        c = lax.axis_index("c")
        s = lax.axis_index("s")
        wid = c * _NS + s

        def _z(j, carry):
            stage[j] = jnp.zeros((16,), jnp.float32)
            return carry

        lax.fori_loop(0, _CH, _z, 0)
        row0 = s * _RPT
        for kk in range(_RPT // _CH):
            pltpu.sync_copy(stage, acc.at[pl.ds(row0 + kk * _CH, _CH)])
        rem = _RPT % _CH
        if rem:
            pltpu.sync_copy(stage.at[pl.ds(0, rem)],
                            acc.at[pl.ds(row0 + _RPT - rem, rem)])
        pltpu.sync_copy(dst_hbm.at[wid], dstb)
        pltpu.sync_copy(w_hbm.at[wid], wb)
        plsc.subcore_barrier()

        def _chunk(ci, carry):
            def _fill(j, c2):
                wj = plsc.load_gather(
                    wb, [jnp.full((16,), ci * _CH + j, jnp.int32)])
                stage[j] = wj
                return c2

            lax.fori_loop(0, _CH, _fill, 0)
            pltpu.sync_copy(stage, acc.at[dstb.at[ci]], add=True)
            return carry

        lax.fori_loop(0, _NCH, _chunk, 0)
        plsc.subcore_barrier()
        pltpu.sync_copy(acc.at[pl.ds(row0, _RPT)], out_hbm.at[wid])

    return k(dstT, wT).reshape(_NC, _N, 16)


def _sc_scatter(g, srcT, dstT, wT):
    @functools.partial(
        pl.kernel,
        out_type=jax.ShapeDtypeStruct((_NW, _RPT, _D), jnp.float32),
        mesh=_sc_mesh(),
        compiler_params=pltpu.CompilerParams(needs_layout_passes=False, use_tc_tiling_on_sc=False),
        scratch_types=[
            pltpu.VMEM_SHARED((_N, _D), jnp.float32),
            pltpu.VMEM((_NCH, _CH), jnp.int32),
            pltpu.VMEM((_NCH, _CH), jnp.int32),
            pltpu.VMEM((_EPT,), jnp.float32),
            pltpu.VMEM((_CH, _D), jnp.float32),
            pltpu.SemaphoreType.DMA,
        ],
    )
    def k(g_hbm, src_hbm, dst_hbm, w_hbm, out_hbm,
          acc, srcb, dstb, wb, rows, sem):
        c = lax.axis_index("c")
        s = lax.axis_index("s")
        wid = c * _NS + s

        def _z(j, carry):
            for t in range(_D // 16):
                rows[j, pl.ds(t * 16, 16)] = jnp.zeros((16,), jnp.float32)
            return carry

        lax.fori_loop(0, _CH, _z, 0)
        row0 = s * _RPT
        for kk in range(_RPT // _CH):
            pltpu.sync_copy(rows, acc.at[pl.ds(row0 + kk * _CH, _CH)])
        rem = _RPT % _CH
        if rem:
            pltpu.sync_copy(rows.at[pl.ds(0, rem)],
                            acc.at[pl.ds(row0 + _RPT - rem, rem)])
        pltpu.sync_copy(src_hbm.at[wid], srcb)
        pltpu.sync_copy(dst_hbm.at[wid], dstb)
        pltpu.sync_copy(w_hbm.at[wid], wb)
        plsc.subcore_barrier()

        def _chunk(ci, carry):
            pltpu.async_copy(g_hbm.at[srcb.at[ci]], rows, sem).wait()

            def _mul(j, c2):
                wj = plsc.load_gather(
                    wb, [jnp.full((16,), ci * _CH + j, jnp.int32)])
                for t in range(_D // 16):
                    rows[j, pl.ds(t * 16, 16)] = (
                        rows[j, pl.ds(t * 16, 16)] * wj)
                return c2

            lax.fori_loop(0, _CH, _mul, 0)
            pltpu.sync_copy(rows, acc.at[dstb.at[ci]], add=True)
            return carry

        lax.fori_loop(0, _NCH, _chunk, 0)
        plsc.subcore_barrier()
        pltpu.sync_copy(acc.at[pl.ds(row0, _RPT)], out_hbm.at[wid])

    return k(g, srcT, dstT, wT).reshape(_NC, _N, _D)


def _tc_matmul(x, W):
    def body(x_ref, w_ref, o_ref):
        o_ref[...] = jnp.dot(x_ref[...], w_ref[...],
                             preferred_element_type=jnp.float32)

    return pl.pallas_call(
        body,
        grid=(_NB,),
        in_specs=[pl.BlockSpec((_BLK, _D), lambda i: (i, 0)),
                  pl.BlockSpec((_D, _D), lambda i: (0, 0))],
        out_specs=pl.BlockSpec((_BLK, _D), lambda i: (i, 0)),
        out_shape=jax.ShapeDtypeStruct((_N, _D), jnp.float32),
    )(x, W)


def _tc_dis_g1(deg2, xw):
    def body(d_ref, xw_ref, dis_ref, g_ref):
        d = d_ref[0] + d_ref[1] + 1.0
        r = lax.rsqrt(d)
        sel = (lax.broadcasted_iota(jnp.int32, (16, _D), 0)
               == lax.broadcasted_iota(jnp.int32, (16, _D), 1) % 16
               ).astype(jnp.float32)
        dis_b = jnp.dot(r, sel, preferred_element_type=jnp.float32,
                        precision=lax.Precision.HIGHEST)
        dis_ref[...] = dis_b
        g_ref[...] = dis_b * xw_ref[...]

    return pl.pallas_call(
        body,
        grid=(_NB,),
        in_specs=[pl.BlockSpec((_NC, _BLK, 16), lambda i: (0, i, 0)),
                  pl.BlockSpec((_BLK, _D), lambda i: (i, 0))],
        out_specs=[pl.BlockSpec((_BLK, _D), lambda i: (i, 0)),
                   pl.BlockSpec((_BLK, _D), lambda i: (i, 0))],
        out_shape=[jax.ShapeDtypeStruct((_N, _D), jnp.float32),
                   jax.ShapeDtypeStruct((_N, _D), jnp.float32)],
    )(deg2, xw)


def _tc_pre_stats(S2, g, dis_b, b):
    def body(s_ref, g_ref, dis_ref, b_ref, h_ref, st_ref):
        i = pl.program_id(0)
        h = dis_ref[...] * (s_ref[0] + s_ref[1] + g_ref[...]) + b_ref[...]
        h_ref[...] = h

        @pl.when(i == 0)
        def _():
            st_ref[...] = jnp.zeros_like(st_ref)

        st_ref[...] += jnp.concatenate(
            [jnp.sum(h, 0, keepdims=True),
             jnp.sum(h * h, 0, keepdims=True)], 0)

    return pl.pallas_call(
        body,
        grid=(_NB,),
        in_specs=[pl.BlockSpec((_NC, _BLK, _D), lambda i: (0, i, 0)),
                  pl.BlockSpec((_BLK, _D), lambda i: (i, 0)),
                  pl.BlockSpec((_BLK, _D), lambda i: (i, 0)),
                  pl.BlockSpec((1, _D), lambda i: (0, 0))],
        out_specs=[pl.BlockSpec((_BLK, _D), lambda i: (i, 0)),
                   pl.BlockSpec((2, _D), lambda i: (0, 0))],
        out_shape=[jax.ShapeDtypeStruct((_N, _D), jnp.float32),
                   jax.ShapeDtypeStruct((2, _D), jnp.float32)],
    )(S2, g, dis_b, b)


def _tc_bn_relu_mm(h, st, gamma, beta, W, dis_b):
    def body(h_ref, st_ref, ga_ref, be_ref, w_ref, dis_ref, o_ref):
        mu = st_ref[0:1] / _N
        var = st_ref[1:2] / _N - mu * mu
        y = ga_ref[...] * (h_ref[...] - mu) * lax.rsqrt(var + 1e-5) + be_ref[...]
        y = jnp.maximum(y, 0.0)
        o_ref[...] = dis_ref[...] * jnp.dot(y, w_ref[...],
                                            preferred_element_type=jnp.float32)

    return pl.pallas_call(
        body,
        grid=(_NB,),
        in_specs=[pl.BlockSpec((_BLK, _D), lambda i: (i, 0)),
                  pl.BlockSpec((2, _D), lambda i: (0, 0)),
                  pl.BlockSpec((1, _D), lambda i: (0, 0)),
                  pl.BlockSpec((1, _D), lambda i: (0, 0)),
                  pl.BlockSpec((_D, _D), lambda i: (0, 0)),
                  pl.BlockSpec((_BLK, _D), lambda i: (i, 0))],
        out_specs=pl.BlockSpec((_BLK, _D), lambda i: (i, 0)),
        out_shape=jax.ShapeDtypeStruct((_N, _D), jnp.float32),
    )(h, st, gamma, beta, W, dis_b)


def _tc_bn(h, st, gamma, beta):
    def body(h_ref, st_ref, ga_ref, be_ref, o_ref):
        mu = st_ref[0:1] / _N
        var = st_ref[1:2] / _N - mu * mu
        o_ref[...] = (ga_ref[...] * (h_ref[...] - mu) * lax.rsqrt(var + 1e-5)
                      + be_ref[...])

    return pl.pallas_call(
        body,
        grid=(_NB,),
        in_specs=[pl.BlockSpec((_BLK, _D), lambda i: (i, 0)),
                  pl.BlockSpec((2, _D), lambda i: (0, 0)),
                  pl.BlockSpec((1, _D), lambda i: (0, 0)),
                  pl.BlockSpec((1, _D), lambda i: (0, 0))],
        out_specs=pl.BlockSpec((_BLK, _D), lambda i: (i, 0)),
        out_shape=jax.ShapeDtypeStruct((_N, _D), jnp.float32),
    )(h, st, gamma, beta)


def kernel(x, edge_index, edge_weight, W1, b1, gamma1, beta1,
           W2, b2, gamma2, beta2):
    src = edge_index[0].reshape(_NW, _NCH, _CH)
    dst = edge_index[1].reshape(_NW, _NCH, _CH)
    w = edge_weight.reshape(_NW, _EPT)

    deg2 = _sc_degree(dst, w)
    xw = _tc_matmul(x, W1)
    dis_b, g1 = _tc_dis_g1(deg2, xw)

    S1 = _sc_scatter(g1, src, dst, w)
    h1, st1 = _tc_pre_stats(S1, g1, dis_b, b1.reshape(1, _D))
    g2 = _tc_bn_relu_mm(h1, st1, gamma1.reshape(1, _D),
                        beta1.reshape(1, _D), W2, dis_b)

    S2 = _sc_scatter(g2, src, dst, w)
    h2, st2 = _tc_pre_stats(S2, g2, dis_b, b2.reshape(1, _D))
    return _tc_bn(h2, st2, gamma2.reshape(1, _D), beta2.reshape(1, _D))


# speedup vs baseline: 13.1926x; 1.0183x over previous
import functools

import jax
import jax.numpy as jnp
from jax import lax
from jax.experimental import pallas as pl
from jax.experimental.pallas import tpu as pltpu
from jax.experimental.pallas import tpu_sc as plsc

_N = 10000
_E = 320000
_D = 128
_NC = 2
_NS = 16
_NW = _NC * _NS
_EPT = _E // _NW
_CH = 50
_NCH = _EPT // _CH
_NBUF = 4
_PF = 2
_RPT = _N // _NS
_BLK = 1000
_NB = _N // _BLK

_SC_PARAMS = pltpu.CompilerParams(needs_layout_passes=False,
                                  use_tc_tiling_on_sc=False)


@functools.lru_cache(maxsize=None)
def _sc_mesh():
    return plsc.VectorSubcoreMesh(core_axis_name="c", subcore_axis_name="s",
                                  num_cores=_NC, num_subcores=_NS)


def _sc_degree(dstT, w16):
    @functools.partial(
        pl.kernel,
        out_type=jax.ShapeDtypeStruct((_NW, _RPT, 16), jnp.float32),
        mesh=_sc_mesh(),
        compiler_params=_SC_PARAMS,
        scratch_types=[
            pltpu.VMEM_SHARED((_N, 16), jnp.float32),
            pltpu.VMEM((_NCH, _CH), jnp.int32),
            pltpu.VMEM((2, _CH, 16), jnp.float32),
            pltpu.SemaphoreType.DMA,
            pltpu.SemaphoreType.DMA,
        ],
    )
    def k(dst_hbm, w16_hbm, out_hbm, acc, dstb, stage, gsem0, gsem1):
        c = lax.axis_index("c")
        s = lax.axis_index("s")
        wid = c * _NS + s
        gsems = (gsem0, gsem1)

        def _z(j, carry):
            stage[0, j] = jnp.zeros((16,), jnp.float32)
            return carry

        lax.fori_loop(0, _CH, _z, 0)
        row0 = s * _RPT
        for kk in range(_RPT // _CH):
            pltpu.sync_copy(stage.at[0], acc.at[pl.ds(row0 + kk * _CH, _CH)])
        rem = _RPT % _CH
        if rem:
            pltpu.sync_copy(stage.at[0, pl.ds(0, rem)],
                            acc.at[pl.ds(row0 + _RPT - rem, rem)])
        pltpu.sync_copy(dst_hbm.at[wid], dstb)
        plsc.subcore_barrier()

        pltpu.async_copy(w16_hbm.at[wid, 0], stage.at[0], gsem0)

        def _chunk(ci, carry):
            for b in range(2):
                cur = ci * 2 + b
                nxt = cur + 1

                @pl.when(nxt < _NCH)
                def _():
                    pltpu.async_copy(w16_hbm.at[wid, nxt],
                                     stage.at[(b + 1) % 2],
                                     gsems[(b + 1) % 2])

                pltpu.make_async_copy(w16_hbm.at[wid, 0], stage.at[b],
                                      gsems[b]).wait()
                pltpu.sync_copy(stage.at[b], acc.at[dstb.at[cur]], add=True)
            return carry

        lax.fori_loop(0, _NCH // 2, _chunk, 0)
        plsc.subcore_barrier()
        pltpu.sync_copy(acc.at[pl.ds(row0, _RPT)], out_hbm.at[wid])

    return k(dstT, w16).reshape(_NC, _N, 16)


def _sc_scatter(g, srcT, dstT, wT):
    @functools.partial(
        pl.kernel,
        out_type=jax.ShapeDtypeStruct((_NW, _RPT, _D), jnp.float32),
        mesh=_sc_mesh(),
        compiler_params=_SC_PARAMS,
        scratch_types=[
            pltpu.VMEM_SHARED((_N, _D), jnp.float32),
            pltpu.VMEM((_NCH, _CH), jnp.int32),
            pltpu.VMEM((_EPT,), jnp.float32),
            pltpu.VMEM((_NBUF, _CH), jnp.int32),
            pltpu.VMEM((_NBUF, _CH, _D), jnp.float32),
            [pltpu.SemaphoreType.DMA] * _NBUF,
            [pltpu.SemaphoreType.DMA] * _NBUF,
            [pltpu.SemaphoreType.DMA] * _NBUF,
        ],
    )
    def k(g_hbm, src_hbm, dst_hbm, w_hbm, out_hbm,
          acc, srcb, wb, dring, rows, gsems, ssems, isems):
        c = lax.axis_index("c")
        s = lax.axis_index("s")
        wid = c * _NS + s

        def _z(j, carry):
            for t in range(_D // 16):
                rows[0, j, pl.ds(t * 16, 16)] = jnp.zeros((16,), jnp.float32)
            return carry

        lax.fori_loop(0, _CH, _z, 0)
        row0 = s * _RPT
        for kk in range(_RPT // _CH):
            pltpu.sync_copy(rows.at[0], acc.at[pl.ds(row0 + kk * _CH, _CH)])
        rem = _RPT % _CH
        if rem:
            pltpu.sync_copy(rows.at[0, pl.ds(0, rem)],
                            acc.at[pl.ds(row0 + _RPT - rem, rem)])
        pltpu.sync_copy(src_hbm.at[wid], srcb)
        pltpu.sync_copy(w_hbm.at[wid], wb)
        plsc.subcore_barrier()

        def _gather_start(ci, b):
            pltpu.async_copy(g_hbm.at[srcb.at[ci]], rows.at[b], gsems[b])

        def _gather_wait(b):
            pltpu.make_async_copy(g_hbm.at[srcb.at[0]], rows.at[b],
                                  gsems[b]).wait()

        def _dst_start(ci, b):
            pltpu.async_copy(dst_hbm.at[wid, ci], dring.at[b], isems[b])

        def _dst_wait(b):
            pltpu.make_async_copy(dst_hbm.at[wid, 0], dring.at[b],
                                  isems[b]).wait()

        def _scat_start(b):
            pltpu.async_copy(rows.at[b], acc.at[dring.at[b]], ssems[b],
                             add=True)

        def _scat_wait(b):
            pltpu.make_async_copy(rows.at[b], acc.at[dring.at[0]],
                                  ssems[b]).wait()

        for ci in range(_PF):
            _dst_start(ci, ci)
            _gather_start(ci, ci)

        def _group(gr, carry):
            for b in range(_NBUF):
                ci = gr * _NBUF + b
                bK = (b + _PF) % _NBUF

                @pl.when(ci >= _PF)
                def _():
                    _scat_wait(bK)

                @pl.when(ci + _PF < _NCH)
                def _():
                    _dst_start(ci + _PF, bK)
                    _gather_start(ci + _PF, bK)

                _gather_wait(b)

                def _mul(j, c2):
                    wj = plsc.load_gather(
                        wb, [jnp.full((16,), ci * _CH + j, jnp.int32)])
                    for t in range(_D // 16):
                        rows[b, j, pl.ds(t * 16, 16)] = (
                            rows[b, j, pl.ds(t * 16, 16)] * wj)
                    return c2

                lax.fori_loop(0, _CH, _mul, 0, unroll=5)
                _dst_wait(b)
                _scat_start(b)
            return carry

        lax.fori_loop(0, _NCH // _NBUF, _group, 0)
        _scat_wait((_NCH - 2) % _NBUF)
        _scat_wait((_NCH - 1) % _NBUF)
        plsc.subcore_barrier()
        pltpu.sync_copy(acc.at[pl.ds(row0, _RPT)], out_hbm.at[wid])

    return k(g, srcT, dstT, wT).reshape(_NC, _N, _D)


def _tc_w16(w):
    def body(w_ref, o_ref):
        o_ref[...] = jnp.broadcast_to(w_ref[...], (8000, 16))

    return pl.pallas_call(
        body,
        grid=(_E // 8000,),
        in_specs=[pl.BlockSpec((8000, 1), lambda i: (i, 0))],
        out_specs=pl.BlockSpec((8000, 16), lambda i: (i, 0)),
        out_shape=jax.ShapeDtypeStruct((_E, 16), jnp.float32),
    )(w)


def _tc_matmul(x, W):
    def body(x_ref, w_ref, o_ref):
        o_ref[...] = jnp.dot(x_ref[...], w_ref[...],
                             preferred_element_type=jnp.float32)

    return pl.pallas_call(
        body,
        grid=(_NB,),
        in_specs=[pl.BlockSpec((_BLK, _D), lambda i: (i, 0)),
                  pl.BlockSpec((_D, _D), lambda i: (0, 0))],
        out_specs=pl.BlockSpec((_BLK, _D), lambda i: (i, 0)),
        out_shape=jax.ShapeDtypeStruct((_N, _D), jnp.float32),
    )(x, W)


def _tc_dis_g1(deg2, xw):
    def body(d_ref, xw_ref, dis_ref, g_ref):
        d = d_ref[0] + d_ref[1] + 1.0
        r = lax.rsqrt(d)
        sel = (lax.broadcasted_iota(jnp.int32, (16, _D), 0)
               == lax.broadcasted_iota(jnp.int32, (16, _D), 1) % 16
               ).astype(jnp.float32)
        dis_b = jnp.dot(r, sel, preferred_element_type=jnp.float32,
                        precision=lax.Precision.HIGHEST)
        dis_ref[...] = dis_b
        g_ref[...] = dis_b * xw_ref[...]

    return pl.pallas_call(
        body,
        grid=(_NB,),
        in_specs=[pl.BlockSpec((_NC, _BLK, 16), lambda i: (0, i, 0)),
                  pl.BlockSpec((_BLK, _D), lambda i: (i, 0))],
        out_specs=[pl.BlockSpec((_BLK, _D), lambda i: (i, 0)),
                   pl.BlockSpec((_BLK, _D), lambda i: (i, 0))],
        out_shape=[jax.ShapeDtypeStruct((_N, _D), jnp.float32),
                   jax.ShapeDtypeStruct((_N, _D), jnp.float32)],
    )(deg2, xw)


def _tc_pre_stats(S2, g, dis_b, b):
    def body(s_ref, g_ref, dis_ref, b_ref, h_ref, st_ref):
        i = pl.program_id(0)
        h = dis_ref[...] * (s_ref[0] + s_ref[1] + g_ref[...]) + b_ref[...]
        h_ref[...] = h

        @pl.when(i == 0)
        def _():
            st_ref[...] = jnp.zeros_like(st_ref)

        st_ref[...] += jnp.concatenate(
            [jnp.sum(h, 0, keepdims=True),
             jnp.sum(h * h, 0, keepdims=True)], 0)

    return pl.pallas_call(
        body,
        grid=(_NB,),
        in_specs=[pl.BlockSpec((_NC, _BLK, _D), lambda i: (0, i, 0)),
                  pl.BlockSpec((_BLK, _D), lambda i: (i, 0)),
                  pl.BlockSpec((_BLK, _D), lambda i: (i, 0)),
                  pl.BlockSpec((1, _D), lambda i: (0, 0))],
        out_specs=[pl.BlockSpec((_BLK, _D), lambda i: (i, 0)),
                   pl.BlockSpec((2, _D), lambda i: (0, 0))],
        out_shape=[jax.ShapeDtypeStruct((_N, _D), jnp.float32),
                   jax.ShapeDtypeStruct((2, _D), jnp.float32)],
    )(S2, g, dis_b, b)


def _tc_bn_relu_mm(h, st, gamma, beta, W, dis_b):
    def body(h_ref, st_ref, ga_ref, be_ref, w_ref, dis_ref, o_ref):
        mu = st_ref[0:1] / _N
        var = st_ref[1:2] / _N - mu * mu
        y = ga_ref[...] * (h_ref[...] - mu) * lax.rsqrt(var + 1e-5) + be_ref[...]
        y = jnp.maximum(y, 0.0)
        o_ref[...] = dis_ref[...] * jnp.dot(y, w_ref[...],
                                            preferred_element_type=jnp.float32)

    return pl.pallas_call(
        body,
        grid=(_NB,),
        in_specs=[pl.BlockSpec((_BLK, _D), lambda i: (i, 0)),
                  pl.BlockSpec((2, _D), lambda i: (0, 0)),
                  pl.BlockSpec((1, _D), lambda i: (0, 0)),
                  pl.BlockSpec((1, _D), lambda i: (0, 0)),
                  pl.BlockSpec((_D, _D), lambda i: (0, 0)),
                  pl.BlockSpec((_BLK, _D), lambda i: (i, 0))],
        out_specs=pl.BlockSpec((_BLK, _D), lambda i: (i, 0)),
        out_shape=jax.ShapeDtypeStruct((_N, _D), jnp.float32),
    )(h, st, gamma, beta, W, dis_b)


def _tc_bn(h, st, gamma, beta):
    def body(h_ref, st_ref, ga_ref, be_ref, o_ref):
        mu = st_ref[0:1] / _N
        var = st_ref[1:2] / _N - mu * mu
        o_ref[...] = (ga_ref[...] * (h_ref[...] - mu) * lax.rsqrt(var + 1e-5)
                      + be_ref[...])

    return pl.pallas_call(
        body,
        grid=(_NB,),
        in_specs=[pl.BlockSpec((_BLK, _D), lambda i: (i, 0)),
                  pl.BlockSpec((2, _D), lambda i: (0, 0)),
                  pl.BlockSpec((1, _D), lambda i: (0, 0)),
                  pl.BlockSpec((1, _D), lambda i: (0, 0))],
        out_specs=pl.BlockSpec((_BLK, _D), lambda i: (i, 0)),
        out_shape=jax.ShapeDtypeStruct((_N, _D), jnp.float32),
    )(h, st, gamma, beta)


def kernel(x, edge_index, edge_weight, W1, b1, gamma1, beta1,
           W2, b2, gamma2, beta2):
    src = edge_index[0].reshape(_NW, _NCH, _CH)
    dst = edge_index[1].reshape(_NW, _NCH, _CH)
    w = edge_weight.reshape(_NW, _EPT)

    w16 = _tc_w16(edge_weight.reshape(_E, 1)).reshape(_NW, _NCH, _CH, 16)
    deg2 = _sc_degree(dst, w16)
    xw = _tc_matmul(x, W1)
    dis_b, g1 = _tc_dis_g1(deg2, xw)

    S1 = _sc_scatter(g1, src, dst, w)
    h1, st1 = _tc_pre_stats(S1, g1, dis_b, b1.reshape(1, _D))
    g2 = _tc_bn_relu_mm(h1, st1, gamma1.reshape(1, _D),
                        beta1.reshape(1, _D), W2, dis_b)

    S2 = _sc_scatter(g2, src, dst, w)
    h2, st2 = _tc_pre_stats(S2, g2, dis_b, b2.reshape(1, _D))
    return _tc_bn(h2, st2, gamma2.reshape(1, _D), beta2.reshape(1, _D))


# speedup vs baseline: 13.9402x; 1.0567x over previous
import functools

import jax
import jax.numpy as jnp
from jax import lax
from jax.experimental import pallas as pl
from jax.experimental.pallas import tpu as pltpu
from jax.experimental.pallas import tpu_sc as plsc

_N = 10000
_E = 320000
_D = 128
_NC = 2
_NS = 16
_NW = _NC * _NS
_EPT = _E // _NW
_CH = 50
_NCH = _EPT // _CH
_NBUF = 4
_PF = 2
_RPT = _N // _NS
_BLK = 1000
_NB = _N // _BLK

_SC_PARAMS = pltpu.CompilerParams(needs_layout_passes=False,
                                  use_tc_tiling_on_sc=False)


@functools.lru_cache(maxsize=None)
def _sc_mesh():
    return plsc.VectorSubcoreMesh(core_axis_name="c", subcore_axis_name="s",
                                  num_cores=_NC, num_subcores=_NS)


_CHD = 100
_NCHD = _EPT // _CHD


def _sc_degree(dstT, w16):
    @functools.partial(
        pl.kernel,
        out_type=jax.ShapeDtypeStruct((_NW, _RPT, 16), jnp.float32),
        mesh=_sc_mesh(),
        compiler_params=_SC_PARAMS,
        scratch_types=[
            pltpu.VMEM_SHARED((_N, 16), jnp.float32),
            pltpu.VMEM((_NCHD, _CHD), jnp.int32),
            pltpu.VMEM((_NBUF, _CHD, 16), jnp.float32),
            [pltpu.SemaphoreType.DMA] * _NBUF,
            [pltpu.SemaphoreType.DMA] * _NBUF,
        ],
    )
    def k(dst_hbm, w16_hbm, out_hbm, acc, dstb, stage, gsems, ssems):
        c = lax.axis_index("c")
        s = lax.axis_index("s")
        wid = c * _NS + s

        def _z(j, carry):
            stage[0, j] = jnp.zeros((16,), jnp.float32)
            return carry

        lax.fori_loop(0, _CHD, _z, 0)
        row0 = s * _RPT
        for kk in range(_RPT // _CHD):
            pltpu.sync_copy(stage.at[0], acc.at[pl.ds(row0 + kk * _CHD, _CHD)])
        rem = _RPT % _CHD
        if rem:
            pltpu.sync_copy(stage.at[0, pl.ds(0, rem)],
                            acc.at[pl.ds(row0 + _RPT - rem, rem)])
        pltpu.sync_copy(dst_hbm.at[wid], dstb)
        plsc.subcore_barrier()

        def _w_start(ci, b):
            pltpu.async_copy(w16_hbm.at[wid, ci], stage.at[b], gsems[b])

        def _w_wait(b):
            pltpu.make_async_copy(w16_hbm.at[wid, 0], stage.at[b],
                                  gsems[b]).wait()

        def _scat_start(ci, b):
            pltpu.async_copy(stage.at[b], acc.at[dstb.at[ci]], ssems[b],
                             add=True)

        def _scat_wait(b):
            pltpu.make_async_copy(stage.at[b], acc.at[dstb.at[0]],
                                  ssems[b]).wait()

        for ci in range(_PF):
            _w_start(ci, ci)

        def _group(gr, carry):
            for b in range(_NBUF):
                ci = gr * _NBUF + b
                bK = (b + _PF) % _NBUF

                @pl.when(ci >= _PF)
                def _():
                    _scat_wait(bK)

                @pl.when(ci + _PF < _NCHD)
                def _():
                    _w_start(ci + _PF, bK)

                _w_wait(b)
                _scat_start(ci, b)
            return carry

        lax.fori_loop(0, _NCHD // _NBUF, _group, 0)
        _scat_wait((_NCHD - 2) % _NBUF)
        _scat_wait((_NCHD - 1) % _NBUF)
        plsc.subcore_barrier()
        pltpu.sync_copy(acc.at[pl.ds(row0, _RPT)], out_hbm.at[wid])

    return k(dstT, w16).reshape(_NC, _N, 16)


def _sc_scatter(g, srcT, dstT, wT):
    @functools.partial(
        pl.kernel,
        out_type=jax.ShapeDtypeStruct((_NW, _RPT, _D), jnp.float32),
        mesh=_sc_mesh(),
        compiler_params=_SC_PARAMS,
        scratch_types=[
            pltpu.VMEM_SHARED((_N, _D), jnp.float32),
            pltpu.VMEM((_NCH, _CH), jnp.int32),
            pltpu.VMEM((_EPT,), jnp.float32),
            pltpu.VMEM((_NBUF, _CH), jnp.int32),
            pltpu.VMEM((_NBUF, _CH, _D), jnp.float32),
            [pltpu.SemaphoreType.DMA] * _NBUF,
            [pltpu.SemaphoreType.DMA] * _NBUF,
            [pltpu.SemaphoreType.DMA] * _NBUF,
        ],
    )
    def k(g_hbm, src_hbm, dst_hbm, w_hbm, out_hbm,
          acc, srcb, wb, dring, rows, gsems, ssems, isems):
        c = lax.axis_index("c")
        s = lax.axis_index("s")
        wid = c * _NS + s

        def _z(j, carry):
            for t in range(_D // 16):
                rows[0, j, pl.ds(t * 16, 16)] = jnp.zeros((16,), jnp.float32)
            return carry

        lax.fori_loop(0, _CH, _z, 0)
        row0 = s * _RPT
        for kk in range(_RPT // _CH):
            pltpu.sync_copy(rows.at[0], acc.at[pl.ds(row0 + kk * _CH, _CH)])
        rem = _RPT % _CH
        if rem:
            pltpu.sync_copy(rows.at[0, pl.ds(0, rem)],
                            acc.at[pl.ds(row0 + _RPT - rem, rem)])
        pltpu.sync_copy(src_hbm.at[wid], srcb)
        pltpu.sync_copy(w_hbm.at[wid], wb)
        plsc.subcore_barrier()

        def _gather_start(ci, b):
            pltpu.async_copy(g_hbm.at[srcb.at[ci]], rows.at[b], gsems[b])

        def _gather_wait(b):
            pltpu.make_async_copy(g_hbm.at[srcb.at[0]], rows.at[b],
                                  gsems[b]).wait()

        def _dst_start(ci, b):
            pltpu.async_copy(dst_hbm.at[wid, ci], dring.at[b], isems[b])

        def _dst_wait(b):
            pltpu.make_async_copy(dst_hbm.at[wid, 0], dring.at[b],
                                  isems[b]).wait()

        def _scat_start(b):
            pltpu.async_copy(rows.at[b], acc.at[dring.at[b]], ssems[b],
                             add=True)

        def _scat_wait(b):
            pltpu.make_async_copy(rows.at[b], acc.at[dring.at[0]],
                                  ssems[b]).wait()

        for ci in range(_PF):
            _dst_start(ci, ci)
            _gather_start(ci, ci)

        def _group(gr, carry):
            for b in range(_NBUF):
                ci = gr * _NBUF + b
                bK = (b + _PF) % _NBUF

                @pl.when(ci >= _PF)
                def _():
                    _scat_wait(bK)

                @pl.when(ci + _PF < _NCH)
                def _():
                    _dst_start(ci + _PF, bK)
                    _gather_start(ci + _PF, bK)

                _gather_wait(b)

                def _mul(j, c2):
                    wj = plsc.load_gather(
                        wb, [jnp.full((16,), ci * _CH + j, jnp.int32)])
                    for t in range(_D // 16):
                        rows[b, j, pl.ds(t * 16, 16)] = (
                            rows[b, j, pl.ds(t * 16, 16)] * wj)
                    return c2

                lax.fori_loop(0, _CH, _mul, 0, unroll=5)
                _dst_wait(b)
                _scat_start(b)
            return carry

        lax.fori_loop(0, _NCH // _NBUF, _group, 0)
        _scat_wait((_NCH - 2) % _NBUF)
        _scat_wait((_NCH - 1) % _NBUF)
        plsc.subcore_barrier()
        pltpu.sync_copy(acc.at[pl.ds(row0, _RPT)], out_hbm.at[wid])

    return k(g, srcT, dstT, wT).reshape(_NC, _N, _D)


def _tc_w16(w):
    def body(w_ref, o_ref):
        o_ref[...] = jnp.broadcast_to(w_ref[...], (8000, 16))

    return pl.pallas_call(
        body,
        grid=(_E // 8000,),
        in_specs=[pl.BlockSpec((8000, 1), lambda i: (i, 0))],
        out_specs=pl.BlockSpec((8000, 16), lambda i: (i, 0)),
        out_shape=jax.ShapeDtypeStruct((_E, 16), jnp.float32),
    )(w)


def _tc_matmul(x, W):
    def body(x_ref, w_ref, o_ref):
        o_ref[...] = jnp.dot(x_ref[...], w_ref[...],
                             preferred_element_type=jnp.float32)

    return pl.pallas_call(
        body,
        grid=(_NB,),
        in_specs=[pl.BlockSpec((_BLK, _D), lambda i: (i, 0)),
                  pl.BlockSpec((_D, _D), lambda i: (0, 0))],
        out_specs=pl.BlockSpec((_BLK, _D), lambda i: (i, 0)),
        out_shape=jax.ShapeDtypeStruct((_N, _D), jnp.float32),
    )(x, W)


def _tc_dis_g1(deg2, xw):
    def body(d_ref, xw_ref, dis_ref, g_ref):
        d = d_ref[0] + d_ref[1] + 1.0
        r = lax.rsqrt(d)
        sel = (lax.broadcasted_iota(jnp.int32, (16, _D), 0)
               == lax.broadcasted_iota(jnp.int32, (16, _D), 1) % 16
               ).astype(jnp.float32)
        dis_b = jnp.dot(r, sel, preferred_element_type=jnp.float32,
                        precision=lax.Precision.HIGHEST)
        dis_ref[...] = dis_b
        g_ref[...] = dis_b * xw_ref[...]

    return pl.pallas_call(
        body,
        grid=(_NB,),
        in_specs=[pl.BlockSpec((_NC, _BLK, 16), lambda i: (0, i, 0)),
                  pl.BlockSpec((_BLK, _D), lambda i: (i, 0))],
        out_specs=[pl.BlockSpec((_BLK, _D), lambda i: (i, 0)),
                   pl.BlockSpec((_BLK, _D), lambda i: (i, 0))],
        out_shape=[jax.ShapeDtypeStruct((_N, _D), jnp.float32),
                   jax.ShapeDtypeStruct((_N, _D), jnp.float32)],
    )(deg2, xw)


def _tc_bn_fused(S2, g, dis_b, b, gamma, beta, W2):
    def body(s_ref, g_ref, dis_ref, b_ref, ga_ref, be_ref, w2_ref,
             o_ref, st):
        i = pl.program_id(0)
        h = dis_ref[...] * (s_ref[0] + s_ref[1] + g_ref[...]) + b_ref[...]

        @pl.when(i == 0)
        def _():
            st[...] = jnp.zeros_like(st)

        @pl.when(i < _NB)
        def _():
            st[0:1] += jnp.sum(h, 0, keepdims=True)
            st[1:2] += jnp.sum(h * h, 0, keepdims=True)

        @pl.when(i >= _NB)
        def _():
            mu = st[0:1] / _N
            var = st[1:2] / _N - mu * mu
            y = (ga_ref[...] * (h - mu) * lax.rsqrt(var + 1e-5)
                 + be_ref[...])
            y = jnp.maximum(y, 0.0)
            o_ref[...] = dis_ref[...] * jnp.dot(
                y, w2_ref[...], preferred_element_type=jnp.float32)

    return pl.pallas_call(
        body,
        grid=(2 * _NB,),
        in_specs=[pl.BlockSpec((_NC, _BLK, _D), lambda i: (0, i % _NB, 0)),
                  pl.BlockSpec((_BLK, _D), lambda i: (i % _NB, 0)),
                  pl.BlockSpec((_BLK, _D), lambda i: (i % _NB, 0)),
                  pl.BlockSpec((1, _D), lambda i: (0, 0)),
                  pl.BlockSpec((1, _D), lambda i: (0, 0)),
                  pl.BlockSpec((1, _D), lambda i: (0, 0)),
                  pl.BlockSpec((_D, _D), lambda i: (0, 0))],
        out_specs=pl.BlockSpec((_BLK, _D), lambda i: (i % _NB, 0)),
        out_shape=jax.ShapeDtypeStruct((_N, _D), jnp.float32),
        scratch_shapes=[pltpu.VMEM((8, _D), jnp.float32)],
    )(S2, g, dis_b, b, gamma, beta, W2)


def _tc_bn_final(S2, g, dis_b, b, gamma, beta):
    def body(s_ref, g_ref, dis_ref, b_ref, ga_ref, be_ref, o_ref, st):
        i = pl.program_id(0)
        h = dis_ref[...] * (s_ref[0] + s_ref[1] + g_ref[...]) + b_ref[...]

        @pl.when(i == 0)
        def _():
            st[...] = jnp.zeros_like(st)

        @pl.when(i < _NB)
        def _():
            st[0:1] += jnp.sum(h, 0, keepdims=True)
            st[1:2] += jnp.sum(h * h, 0, keepdims=True)

        @pl.when(i >= _NB)
        def _():
            mu = st[0:1] / _N
            var = st[1:2] / _N - mu * mu
            o_ref[...] = (ga_ref[...] * (h - mu) * lax.rsqrt(var + 1e-5)
                          + be_ref[...])

    return pl.pallas_call(
        body,
        grid=(2 * _NB,),
        in_specs=[pl.BlockSpec((_NC, _BLK, _D), lambda i: (0, i % _NB, 0)),
                  pl.BlockSpec((_BLK, _D), lambda i: (i % _NB, 0)),
                  pl.BlockSpec((_BLK, _D), lambda i: (i % _NB, 0)),
                  pl.BlockSpec((1, _D), lambda i: (0, 0)),
                  pl.BlockSpec((1, _D), lambda i: (0, 0)),
                  pl.BlockSpec((1, _D), lambda i: (0, 0))],
        out_specs=pl.BlockSpec((_BLK, _D), lambda i: (i % _NB, 0)),
        out_shape=jax.ShapeDtypeStruct((_N, _D), jnp.float32),
        scratch_shapes=[pltpu.VMEM((8, _D), jnp.float32)],
    )(S2, g, dis_b, b, gamma, beta)


def kernel(x, edge_index, edge_weight, W1, b1, gamma1, beta1,
           W2, b2, gamma2, beta2):
    src = edge_index[0].reshape(_NW, _NCH, _CH)
    dst = edge_index[1].reshape(_NW, _NCH, _CH)
    dst_d = edge_index[1].reshape(_NW, _NCHD, _CHD)
    w = edge_weight.reshape(_NW, _EPT)

    w16 = _tc_w16(edge_weight.reshape(_E, 1)).reshape(_NW, _NCHD, _CHD, 16)
    deg2 = _sc_degree(dst_d, w16)
    xw = _tc_matmul(x, W1)
    dis_b, g1 = _tc_dis_g1(deg2, xw)

    S1 = _sc_scatter(g1, src, dst, w)
    g2 = _tc_bn_fused(S1, g1, dis_b, b1.reshape(1, _D),
                      gamma1.reshape(1, _D), beta1.reshape(1, _D), W2)

    S2 = _sc_scatter(g2, src, dst, w)
    return _tc_bn_final(S2, g2, dis_b, b2.reshape(1, _D),
                        gamma2.reshape(1, _D), beta2.reshape(1, _D))
